# baseline (speedup 1.0000x reference)
import functools

import jax
import jax.numpy as jnp
from jax import lax
from jax.experimental import pallas as pl
from jax.experimental.pallas import tpu as pltpu
from jax.experimental.pallas import tpu_sc as plsc

T, N, E = 12, 10000, 320000
IN, H = 24, 12

NC, NS, L = 2, 16, 16
SF = 2
NG = T // SF
GPC = NG // NC
CH = 128
CPT = 160
EPT = CPT * CH
E_PAD = NS * EPT
RPT = 632
N_PAD = NS * RPT
LAST_ROWS = N - (NS - 1) * RPT

NBUF = 4
GROUPS = CPT // NBUF


def _sc_body(x_hbm, src_hbm, dst_hbm, zero_hbm, out_hbm,
             src_v, dst_v, rows_v, agg_sh, x_sh,
             sg0, sg1, sg2, sg3, ss0, ss1, ss2, ss3):
    c = lax.axis_index("c")
    w = lax.axis_index("s")

    pltpu.sync_copy(src_hbm.at[pl.ds(w * EPT, EPT)], src_v)
    pltpu.sync_copy(dst_hbm.at[pl.ds(w * CPT, CPT)], dst_v)

    def group_body(i, carry):
        g = c * GPC + i

        pltpu.sync_copy(zero_hbm, agg_sh.at[pl.ds(w * RPT, RPT)])

        my_lo = w * RPT

        @pl.when(w == NS - 1)
        def _():
            for s in range(SF):
                pltpu.sync_copy(
                    x_hbm.at[pl.ds((SF * g + s) * N + my_lo, LAST_ROWS)],
                    x_sh.at[pl.ds(my_lo, LAST_ROWS), pl.ds(s * IN, IN)])

        @pl.when(w != NS - 1)
        def _():
            for s in range(SF):
                pltpu.sync_copy(
                    x_hbm.at[pl.ds((SF * g + s) * N + my_lo, RPT)],
                    x_sh.at[pl.ds(my_lo, RPT), pl.ds(s * IN, IN)])

        plsc.subcore_barrier()

        sg = (sg0, sg1, sg2, sg3)
        ss = (ss0, ss1, ss2, ss3)

        def src_idx(j):
            return src_v.at[pl.ds(j * CH, CH)]

        def start_gather(slot, j):
            pltpu.async_copy(x_sh.at[src_idx(j)], rows_v.at[slot], sg[slot])

        def drain_gather(slot, j):
            pltpu.make_async_copy(x_sh.at[src_idx(j)],
                                  rows_v.at[slot], sg[slot]).wait()

        def start_scatter(slot, j):
            pltpu.async_copy(rows_v.at[slot], agg_sh.at[dst_v.at[j]],
                             ss[slot], add=True)

        def drain_scatter(slot, j):
            pltpu.make_async_copy(rows_v.at[slot], agg_sh.at[dst_v.at[j]],
                                  ss[slot]).wait()

        start_gather(0, 0)
        start_gather(1, 1)

        def pipe_body(jj, carry2):
            for b in range(NBUF):
                j = NBUF * jj + b
                b2 = (b + 2) % NBUF
                drain_gather(b, j)
                start_scatter(b, j)
                if b < 2:
                    @pl.when(jj > 0)
                    def _():
                        drain_scatter(b2, j)
                    start_gather(b2, j + 2)
                else:
                    drain_scatter(b2, j)

                    @pl.when(jj < GROUPS - 1)
                    def _():
                        start_gather(b2, j + 2)
            return carry2

        lax.fori_loop(0, GROUPS, pipe_body, 0)
        drain_scatter(2, CPT - 2)
        drain_scatter(3, CPT - 1)
        plsc.subcore_barrier()

        @pl.when(w == NS - 1)
        def _():
            pltpu.sync_copy(
                agg_sh.at[pl.ds((NS - 1) * RPT, LAST_ROWS)],
                out_hbm.at[pl.ds(g * N + (NS - 1) * RPT, LAST_ROWS)])

        @pl.when(w != NS - 1)
        def _():
            pltpu.sync_copy(agg_sh.at[pl.ds(w * RPT, RPT)],
                            out_hbm.at[pl.ds(g * N + w * RPT, RPT)])

        return carry

    lax.fori_loop(0, GPC, group_body, 0)


def _segment_sums(x_rows, src_pad, dst_pad, zeros):
    mesh = plsc.VectorSubcoreMesh(core_axis_name="c", subcore_axis_name="s")
    return pl.kernel(
        _sc_body,
        out_type=jax.ShapeDtypeStruct((NG * N, SF * IN), jnp.float32),
        mesh=mesh,
        scratch_types=[
            pltpu.VMEM((EPT,), jnp.int32),
            pltpu.VMEM((CPT, CH), jnp.int32),
            pltpu.VMEM((NBUF, CH, SF * IN), jnp.float32),
            pltpu.VMEM_SHARED((N_PAD, SF * IN), jnp.float32),
            pltpu.VMEM_SHARED((N, SF * IN), jnp.float32),
        ] + [pltpu.SemaphoreType.DMA] * (2 * NBUF),
        compiler_params=pltpu.CompilerParams(use_tc_tiling_on_sc=False),
    )(x_rows, src_pad, dst_pad, zeros)


ROWS_BLK = 2000
NBLK = N // ROWS_BLK


def _tc_body(x_ref, a_ref, wzx_ref, wza_ref, wnx_ref, wna_ref, bz_ref,
             bn_ref, w1_ref, b1_ref, w2_ref, b2_ref, o_ref):
    dot = functools.partial(jnp.dot, preferred_element_type=jnp.float32)
    xb = x_ref[...]
    s = lax.rem(pl.program_id(0), SF)
    a2 = a_ref[...]
    ab = jnp.where(s == 0, a2[:, :IN], a2[:, IN:])
    z = jax.nn.sigmoid(dot(xb, wzx_ref[...]) + dot(ab, wza_ref[...])
                       + bz_ref[...])
    n = jnp.tanh(dot(xb, wnx_ref[...]) + dot(ab, wna_ref[...]) + bn_ref[...])
    h = jax.nn.relu((1.0 - z) * n)
    h = jax.nn.relu(dot(h, w1_ref[...]) + b1_ref[...])
    o_ref[...] = jax.nn.sigmoid(dot(h, w2_ref[...]) + b2_ref[...])


def _dense_head(x_flat, agg, Az, Bz, An, Bn, bz, bn, W1, b1, W2, b2):
    grid = (T, NBLK)
    x_spec = pl.BlockSpec((ROWS_BLK, IN), lambda t, i: (t * NBLK + i, 0))
    a_spec = pl.BlockSpec(
        (ROWS_BLK, SF * IN),
        lambda t, i: (lax.div(t, SF) * NBLK + i, 0))
    w_spec = pl.BlockSpec((IN, H), lambda t, i: (0, 0))
    h_spec = pl.BlockSpec((H, H), lambda t, i: (0, 0))
    b_spec = pl.BlockSpec((1, H), lambda t, i: (0, 0))
    return pl.pallas_call(
        _tc_body,
        grid=grid,
        in_specs=[x_spec, a_spec, w_spec, w_spec, w_spec, w_spec,
                  b_spec, b_spec, h_spec, b_spec, h_spec, b_spec],
        out_specs=pl.BlockSpec((ROWS_BLK, H), lambda t, i: (t * NBLK + i, 0)),
        out_shape=jax.ShapeDtypeStruct((T * N, H), jnp.float32),
    )(x_flat, agg, Az, Bz, An, Bn, bz, bn, W1, b1, W2, b2)


def kernel(x, edge_index, Wz_root, Wz_agg, Wr_root, Wr_agg, Wn_root, Wn_agg,
           bz, br, bn, W1, b1, W2, b2):
    src = edge_index[0]
    dst = edge_index[1]
    src_pad = jnp.concatenate(
        [src, jnp.zeros((E_PAD - E,), jnp.int32)])
    dst_pad = jnp.concatenate(
        [dst, jnp.full((E_PAD - E,), N, jnp.int32)]).reshape(E_PAD // CH, CH)
    x_rows = x.reshape(T * N, IN)
    zeros = jnp.zeros((RPT, SF * IN), jnp.float32)

    agg = _segment_sums(x_rows, src_pad, dst_pad, zeros)

    out = _dense_head(
        x.reshape(T * N, IN), agg,
        Wz_root[:IN], Wz_agg[:IN], Wn_root[:IN], Wn_agg[:IN],
        bz.reshape(1, H), bn.reshape(1, H),
        W1, b1.reshape(1, H), W2, b2.reshape(1, H))
    return out.reshape(T, N, H)


# speedup vs baseline: 1.0310x; 1.0310x over previous
import functools

import jax
import jax.numpy as jnp
from jax import lax
from jax.experimental import pallas as pl
from jax.experimental.pallas import tpu as pltpu
from jax.experimental.pallas import tpu_sc as plsc

T, N, E = 12, 10000, 320000
IN, H = 24, 12

NC, NS, L = 2, 16, 16
SF = 2
NG = T // SF
GPC = NG // NC
CH = 128
E_CHUNKS = E // CH
CPT_U = E_CHUNKS // NS
XTRA = E_CHUNKS - NS * CPT_U
CPT_MAX = CPT_U + 1
EPT_MAX = CPT_MAX * CH
RPT = 632
N_PAD = NS * RPT
LAST_ROWS = N - (NS - 1) * RPT

NBUF = 4
GROUPS = CPT_U // NBUF


def _sc_body(x_hbm, src_hbm, dst_hbm, zero_hbm, out_hbm,
             src_v, dst_v, rows_v, agg_sh, x_sh,
             sg0, sg1, sg2, sg3, ss0, ss1, ss2, ss3):
    c = lax.axis_index("c")
    w = lax.axis_index("s")

    coff = w * CPT_U + jnp.minimum(w, XTRA)

    @pl.when(w < XTRA)
    def _():
        pltpu.sync_copy(src_hbm.at[pl.ds(coff * CH, CPT_MAX * CH)],
                        src_v.at[pl.ds(0, CPT_MAX * CH)])
        pltpu.sync_copy(dst_hbm.at[pl.ds(coff, CPT_MAX)],
                        dst_v.at[pl.ds(0, CPT_MAX)])

    @pl.when(w >= XTRA)
    def _():
        pltpu.sync_copy(src_hbm.at[pl.ds(coff * CH, CPT_U * CH)],
                        src_v.at[pl.ds(0, CPT_U * CH)])
        pltpu.sync_copy(dst_hbm.at[pl.ds(coff, CPT_U)],
                        dst_v.at[pl.ds(0, CPT_U)])

    def group_body(i, carry):
        g = c * GPC + i

        pltpu.sync_copy(zero_hbm, agg_sh.at[pl.ds(w * RPT, RPT)])

        my_lo = w * RPT

        @pl.when(w == NS - 1)
        def _():
            for s in range(SF):
                pltpu.sync_copy(
                    x_hbm.at[pl.ds((SF * g + s) * N + my_lo, LAST_ROWS)],
                    x_sh.at[pl.ds(my_lo, LAST_ROWS), pl.ds(s * IN, IN)])

        @pl.when(w != NS - 1)
        def _():
            for s in range(SF):
                pltpu.sync_copy(
                    x_hbm.at[pl.ds((SF * g + s) * N + my_lo, RPT)],
                    x_sh.at[pl.ds(my_lo, RPT), pl.ds(s * IN, IN)])

        plsc.subcore_barrier()

        sg = (sg0, sg1, sg2, sg3)
        ss = (ss0, ss1, ss2, ss3)

        def src_idx(j):
            return src_v.at[pl.ds(j * CH, CH)]

        def start_gather(slot, j):
            pltpu.async_copy(x_sh.at[src_idx(j)], rows_v.at[slot], sg[slot])

        def drain_gather(slot, j):
            pltpu.make_async_copy(x_sh.at[src_idx(j)],
                                  rows_v.at[slot], sg[slot]).wait()

        def start_scatter(slot, j):
            pltpu.async_copy(rows_v.at[slot], agg_sh.at[dst_v.at[j]],
                             ss[slot], add=True)

        def drain_scatter(slot, j):
            pltpu.make_async_copy(rows_v.at[slot], agg_sh.at[dst_v.at[j]],
                                  ss[slot]).wait()

        start_gather(0, 0)
        start_gather(1, 1)

        def pipe_body(jj, carry2):
            for b in range(NBUF):
                j = NBUF * jj + b
                b2 = (b + 2) % NBUF
                drain_gather(b, j)
                start_scatter(b, j)
                if b < 2:
                    @pl.when(jj > 0)
                    def _():
                        drain_scatter(b2, j)
                    start_gather(b2, j + 2)
                else:
                    drain_scatter(b2, j)

                    @pl.when(jj < GROUPS - 1)
                    def _():
                        start_gather(b2, j + 2)
            return carry2

        lax.fori_loop(0, GROUPS, pipe_body, 0)
        drain_scatter(2, CPT_U - 2)
        drain_scatter(3, CPT_U - 1)

        @pl.when(w < XTRA)
        def _():
            start_gather(0, CPT_U)
            drain_gather(0, CPT_U)
            start_scatter(0, CPT_U)
            drain_scatter(0, CPT_U)

        plsc.subcore_barrier()

        @pl.when(w == NS - 1)
        def _():
            pltpu.sync_copy(
                agg_sh.at[pl.ds((NS - 1) * RPT, LAST_ROWS)],
                out_hbm.at[pl.ds(g * N + (NS - 1) * RPT, LAST_ROWS)])

        @pl.when(w != NS - 1)
        def _():
            pltpu.sync_copy(agg_sh.at[pl.ds(w * RPT, RPT)],
                            out_hbm.at[pl.ds(g * N + w * RPT, RPT)])

        return carry

    lax.fori_loop(0, GPC, group_body, 0)


def _segment_sums(x_rows, src_pad, dst_pad, zeros):
    mesh = plsc.VectorSubcoreMesh(core_axis_name="c", subcore_axis_name="s")
    return pl.kernel(
        _sc_body,
        out_type=jax.ShapeDtypeStruct((NG * N, SF * IN), jnp.float32),
        mesh=mesh,
        scratch_types=[
            pltpu.VMEM((EPT_MAX,), jnp.int32),
            pltpu.VMEM((CPT_MAX, CH), jnp.int32),
            pltpu.VMEM((NBUF, CH, SF * IN), jnp.float32),
            pltpu.VMEM_SHARED((N_PAD, SF * IN), jnp.float32),
            pltpu.VMEM_SHARED((N, SF * IN), jnp.float32),
        ] + [pltpu.SemaphoreType.DMA] * (2 * NBUF),
        compiler_params=pltpu.CompilerParams(use_tc_tiling_on_sc=False),
    )(x_rows, src_pad, dst_pad, zeros)


ROWS_BLK = 2000
NBLK = N // ROWS_BLK


def _tc_body(x_ref, a_ref, wzx_ref, wza_ref, wnx_ref, wna_ref, bz_ref,
             bn_ref, w1_ref, b1_ref, w2_ref, b2_ref, o_ref):
    dot = functools.partial(jnp.dot, preferred_element_type=jnp.float32)
    xb = x_ref[...]
    s = lax.rem(pl.program_id(0), SF)
    a2 = a_ref[...]
    ab = jnp.where(s == 0, a2[:, :IN], a2[:, IN:])
    z = jax.nn.sigmoid(dot(xb, wzx_ref[...]) + dot(ab, wza_ref[...])
                       + bz_ref[...])
    n = jnp.tanh(dot(xb, wnx_ref[...]) + dot(ab, wna_ref[...]) + bn_ref[...])
    h = jax.nn.relu((1.0 - z) * n)
    h = jax.nn.relu(dot(h, w1_ref[...]) + b1_ref[...])
    o_ref[...] = jax.nn.sigmoid(dot(h, w2_ref[...]) + b2_ref[...])


def _dense_head(x_flat, agg, Az, Bz, An, Bn, bz, bn, W1, b1, W2, b2):
    grid = (T, NBLK)
    x_spec = pl.BlockSpec((ROWS_BLK, IN), lambda t, i: (t * NBLK + i, 0))
    a_spec = pl.BlockSpec(
        (ROWS_BLK, SF * IN),
        lambda t, i: (lax.div(t, SF) * NBLK + i, 0))
    w_spec = pl.BlockSpec((IN, H), lambda t, i: (0, 0))
    h_spec = pl.BlockSpec((H, H), lambda t, i: (0, 0))
    b_spec = pl.BlockSpec((1, H), lambda t, i: (0, 0))
    return pl.pallas_call(
        _tc_body,
        grid=grid,
        in_specs=[x_spec, a_spec, w_spec, w_spec, w_spec, w_spec,
                  b_spec, b_spec, h_spec, b_spec, h_spec, b_spec],
        out_specs=pl.BlockSpec((ROWS_BLK, H), lambda t, i: (t * NBLK + i, 0)),
        out_shape=jax.ShapeDtypeStruct((T * N, H), jnp.float32),
    )(x_flat, agg, Az, Bz, An, Bn, bz, bn, W1, b1, W2, b2)


def kernel(x, edge_index, Wz_root, Wz_agg, Wr_root, Wr_agg, Wn_root, Wn_agg,
           bz, br, bn, W1, b1, W2, b2):
    src = edge_index[0]
    dst = edge_index[1].reshape(E_CHUNKS, CH)
    x_rows = x.reshape(T * N, IN)
    zeros = jnp.zeros((RPT, SF * IN), jnp.float32)

    agg = _segment_sums(x_rows, src, dst, zeros)

    out = _dense_head(
        x.reshape(T * N, IN), agg,
        Wz_root[:IN], Wz_agg[:IN], Wn_root[:IN], Wn_agg[:IN],
        bz.reshape(1, H), bn.reshape(1, H),
        W1, b1.reshape(1, H), W2, b2.reshape(1, H))
    return out.reshape(T, N, H)
